# baseline (speedup 1.0000x reference)
import functools

import jax
import jax.numpy as jnp
from jax import lax
from jax.experimental import pallas as pl
from jax.experimental.pallas import tpu as pltpu
from jax.experimental.pallas import tpu_sc as plsc

VOCAB = 100000
EMBED = 64
MAX_WORD_LEN = 50
HIDDEN = 1024
N_CLASSES = 128
BATCH = 4096

IN_DIM = MAX_WORD_LEN * EMBED


def _sc_worker_count():
    try:
        info = plsc.get_sparse_core_info()
        return info.num_cores * info.num_subcores
    except Exception:
        return 32


@functools.lru_cache(maxsize=None)
def _make_sc_gather(n_workers: int):
    e_per_w = EMBED // n_workers
    mesh = plsc.VectorSubcoreMesh(core_axis_name="c", subcore_axis_name="s")

    @functools.partial(
        pl.kernel,
        out_type=jax.ShapeDtypeStruct((IN_DIM, BATCH), jnp.float32),
        mesh=mesh,
        scratch_types=[
            pltpu.VMEM((VOCAB,), jnp.float32),
            pltpu.VMEM((BATCH,), jnp.int32),
            pltpu.VMEM((BATCH,), jnp.float32),
            pltpu.SemaphoreType.DMA,
        ],
        compiler_params=pltpu.CompilerParams(
            use_tc_tiling_on_sc=False, needs_layout_passes=False),
    )
    def gather_kernel(xt_hbm, table_hbm, out_hbm, trow_v, idx_v, obuf_v, sem):
        wid = lax.axis_index("s") * 2 + lax.axis_index("c")

        def do_lane(l, e):
            pltpu.sync_copy(xt_hbm.at[pl.ds(l * BATCH, BATCH)], idx_v)

            def body(j, _):
                iv = idx_v[pl.ds(j * 16, 16)]
                obuf_v[pl.ds(j * 16, 16)] = plsc.load_gather(trow_v, [iv])
                return 0

            lax.fori_loop(0, BATCH // 16, body, 0, unroll=8)
            pltpu.sync_copy(obuf_v, out_hbm.at[l * EMBED + e])
            return 0

        for ep in range(e_per_w):
            e = wid * e_per_w + ep
            pltpu.sync_copy(
                table_hbm.at[pl.ds(pl.multiple_of(e * VOCAB, 8), VOCAB)],
                trow_v)
            lax.fori_loop(0, MAX_WORD_LEN, lambda l, _, e=e: do_lane(l, e), 0)

    return gather_kernel


def _mlp_body(a_ref, w1_ref, b1_ref, w2_ref, b2_ref, out_ref):
    h = lax.dot_general(w1_ref[...], a_ref[...], (((1,), (0,)), ((), ())),
                        preferred_element_type=jnp.float32)
    h = jnp.maximum(h + b1_ref[...], 0.0)
    out_ref[...] = lax.dot_general(w2_ref[...], h, (((1,), (0,)), ((), ())),
                                   preferred_element_type=jnp.float32) + b2_ref[...]


@functools.lru_cache(maxsize=None)
def _make_mlp(tile_b: int):
    grid = (BATCH // tile_b,)
    return pl.pallas_call(
        _mlp_body,
        grid=grid,
        in_specs=[
            pl.BlockSpec((IN_DIM, tile_b), lambda i: (0, i)),
            pl.BlockSpec((HIDDEN, IN_DIM), lambda i: (0, 0)),
            pl.BlockSpec((HIDDEN, 1), lambda i: (0, 0)),
            pl.BlockSpec((N_CLASSES, HIDDEN), lambda i: (0, 0)),
            pl.BlockSpec((N_CLASSES, 1), lambda i: (0, 0)),
        ],
        out_specs=pl.BlockSpec((N_CLASSES, tile_b), lambda i: (0, i)),
        out_shape=jax.ShapeDtypeStruct((N_CLASSES, BATCH), jnp.float32),
    )


def kernel(x, table, W1, b1, W2, b2):
    nw = _sc_worker_count()
    xt = x.astype(jnp.int32).T.reshape(-1)
    tflat = table.T.reshape(-1)
    a_cm = _make_sc_gather(nw)(xt, tflat)
    out_cm = _make_mlp(512)(
        a_cm, W1, b1.reshape(HIDDEN, 1), W2, b2.reshape(N_CLASSES, 1))
    return out_cm.T


# speedup vs baseline: 3.1155x; 3.1155x over previous
import functools

import jax
import jax.numpy as jnp
from jax import lax
from jax.experimental import pallas as pl
from jax.experimental.pallas import tpu as pltpu
from jax.experimental.pallas import tpu_sc as plsc

VOCAB = 100000
EMBED = 64
MAX_WORD_LEN = 50
HIDDEN = 1024
N_CLASSES = 128
BATCH = 4096

NCHUNK = 2
BCH = BATCH // NCHUNK
PAIRS = MAX_WORD_LEN // 2
YROWS = BCH * MAX_WORD_LEN // 2
IDX_COLS = 12800


def _sc_worker_count():
    try:
        info = plsc.get_sparse_core_info()
        return info.num_cores * info.num_subcores
    except Exception:
        return 32


@functools.lru_cache(maxsize=None)
def _make_sc_gather(n_workers: int, chunk: int):
    per_w = YROWS // n_workers
    n_chunks = per_w // chunk
    mesh = plsc.VectorSubcoreMesh(core_axis_name="c", subcore_axis_name="s")

    @functools.partial(
        pl.kernel,
        out_type=jax.ShapeDtypeStruct((YROWS, 2 * EMBED), jnp.float32),
        mesh=mesh,
        scratch_types=[
            pltpu.VMEM((2, chunk), jnp.int32),
            pltpu.VMEM((2, chunk, EMBED), jnp.float32),
            pltpu.SemaphoreType.DMA,
            pltpu.SemaphoreType.DMA,
            pltpu.SemaphoreType.DMA,
            pltpu.SemaphoreType.DMA,
        ],
        compiler_params=pltpu.CompilerParams(use_tc_tiling_on_sc=False),
    )
    def gather_kernel(idx_hbm, table_hbm, out_hbm, idx_v, rows_v, g0, g1, w0, w1):
        wid = lax.axis_index("s") * 2 + lax.axis_index("c")
        base = wid * per_w
        units = [(c, h) for c in range(n_chunks) for h in range(2)]
        gsem = (g0, g1)
        wsem = (w0, w1)

        def load_idx_start_gather(u):
            c, h = units[u]
            flat = h * YROWS + base + c * chunk
            pltpu.sync_copy(
                idx_hbm.at[flat // IDX_COLS, pl.ds(flat % IDX_COLS, chunk)],
                idx_v.at[u % 2])
            return pltpu.async_copy(
                table_hbm.at[idx_v.at[u % 2]], rows_v.at[u % 2], gsem[u % 2])

        def start_scatter(u):
            c, h = units[u]
            off = base + c * chunk
            return pltpu.async_copy(
                rows_v.at[u % 2],
                out_hbm.at[pl.ds(off, chunk), pl.ds(h * EMBED, EMBED)],
                wsem[u % 2])

        n = len(units)
        gops = [None] * n
        wops = [None] * n
        gops[0] = load_idx_start_gather(0)
        for u in range(n):
            if u + 1 < n:
                if u >= 1:
                    wops[u - 1].wait()
                gops[u + 1] = load_idx_start_gather(u + 1)
            gops[u].wait()
            wops[u] = start_scatter(u)
        wops[n - 2].wait()
        wops[n - 1].wait()

    return gather_kernel


def _mlp_body(z_ref, w1_ref, b1_ref, w2_ref, b2_ref, out_ref):
    z = z_ref[...]
    a = jnp.concatenate([z[t] for t in range(PAIRS)], axis=1)
    h = lax.dot_general(a, w1_ref[...], (((1,), (1,)), ((), ())),
                        preferred_element_type=jnp.float32)
    h = jnp.maximum(h + b1_ref[...], 0.0)
    out_ref[...] = lax.dot_general(h, w2_ref[...], (((1,), (1,)), ((), ())),
                                   preferred_element_type=jnp.float32) + b2_ref[...]


@functools.lru_cache(maxsize=None)
def _make_mlp(tile_b: int):
    in_dim = MAX_WORD_LEN * EMBED
    grid = (BCH // tile_b,)
    return pl.pallas_call(
        _mlp_body,
        grid=grid,
        in_specs=[
            pl.BlockSpec((PAIRS, tile_b, 2 * EMBED), lambda i: (0, i, 0)),
            pl.BlockSpec((HIDDEN, in_dim), lambda i: (0, 0)),
            pl.BlockSpec((1, HIDDEN), lambda i: (0, 0)),
            pl.BlockSpec((N_CLASSES, HIDDEN), lambda i: (0, 0)),
            pl.BlockSpec((1, N_CLASSES), lambda i: (0, 0)),
        ],
        out_specs=pl.BlockSpec((tile_b, N_CLASSES), lambda i: (i, 0)),
        out_shape=jax.ShapeDtypeStruct((BCH, N_CLASSES), jnp.float32),
    )


def kernel(x, table, W1, b1, W2, b2):
    nw = _sc_worker_count()
    xi = x.astype(jnp.int32)
    b1r = b1.reshape(1, HIDDEN)
    b2r = b2.reshape(1, N_CLASSES)
    outs = []
    for c in range(NCHUNK):
        xc = xi[c * BCH:(c + 1) * BCH]
        idx2 = xc.reshape(BCH, PAIRS, 2).transpose(2, 1, 0)
        idx16 = idx2.reshape(2 * YROWS // IDX_COLS, IDX_COLS)
        y2 = _make_sc_gather(nw, 800)(idx16, table)
        z = y2.reshape(PAIRS, BCH, 2 * EMBED)
        outs.append(_make_mlp(512)(z, W1, b1r, W2, b2r))
    return jnp.concatenate(outs, axis=0)
